# baseline (speedup 1.0000x reference)
import jax
import jax.numpy as jnp
from jax.experimental import pallas as pl
from jax.experimental.pallas import tpu as pltpu

_BM = 512


def _linear_kernel(x_ref, w_ref, b_ref, o_ref):
    acc = jax.lax.dot_general(
        x_ref[...],
        w_ref[...],
        dimension_numbers=(((1,), (1,)), ((), ())),
        preferred_element_type=jnp.float32,
    )
    o_ref[...] = acc + b_ref[...]


def kernel(src, W, b):
    B, S, D = src.shape
    E = W.shape[0]
    M = B * S
    x = src.reshape(M, D)

    out = pl.pallas_call(
        _linear_kernel,
        grid=(M // _BM,),
        in_specs=[
            pl.BlockSpec((_BM, D), lambda i: (i, 0)),
            pl.BlockSpec((E, D), lambda i: (0, 0)),
            pl.BlockSpec((1, E), lambda i: (0, 0)),
        ],
        out_specs=pl.BlockSpec((_BM, E), lambda i: (i, 0)),
        out_shape=jax.ShapeDtypeStruct((M, E), jnp.float32),
        compiler_params=pltpu.CompilerParams(
            dimension_semantics=("parallel",),
        ),
    )(x, W, b.reshape(1, E))
    return out.reshape(B, S, E)


# speedup vs baseline: 1.2292x; 1.2292x over previous
import jax
import jax.numpy as jnp
from jax.experimental import pallas as pl
from jax.experimental.pallas import tpu as pltpu

_BM = 1024


def _linear_kernel(x_ref, w_ref, b_ref, o_ref):
    acc = jax.lax.dot_general(
        x_ref[...],
        w_ref[...],
        dimension_numbers=(((1,), (0,)), ((), ())),
        preferred_element_type=jnp.float32,
    )
    o_ref[...] = acc + b_ref[...]


def kernel(src, W, b):
    B, S, D = src.shape
    E = W.shape[0]
    M = B * S
    x = src.reshape(M, D)

    out = pl.pallas_call(
        _linear_kernel,
        grid=(M // _BM,),
        in_specs=[
            pl.BlockSpec((_BM, D), lambda i: (i, 0)),
            pl.BlockSpec((D, E), lambda i: (0, 0)),
            pl.BlockSpec((1, E), lambda i: (0, 0)),
        ],
        out_specs=pl.BlockSpec((_BM, E), lambda i: (i, 0)),
        out_shape=jax.ShapeDtypeStruct((M, E), jnp.float32),
        compiler_params=pltpu.CompilerParams(
            dimension_semantics=("parallel",),
        ),
    )(x, W.T, b.reshape(1, E))
    return out.reshape(B, S, E)


# speedup vs baseline: 1.3857x; 1.1273x over previous
import jax
import jax.numpy as jnp
from jax.experimental import pallas as pl
from jax.experimental.pallas import tpu as pltpu

_BM = 2048


def _linear_kernel(x_ref, w_ref, b_ref, o_ref):
    acc = jax.lax.dot_general(
        x_ref[...],
        w_ref[...],
        dimension_numbers=(((1,), (0,)), ((), ())),
        preferred_element_type=jnp.float32,
    )
    o_ref[...] = acc + b_ref[...]


def kernel(src, W, b):
    B, S, D = src.shape
    E = W.shape[0]
    M = B * S
    x = src.reshape(M, D)

    out = pl.pallas_call(
        _linear_kernel,
        grid=(M // _BM,),
        in_specs=[
            pl.BlockSpec((_BM, D), lambda i: (i, 0)),
            pl.BlockSpec((D, E), lambda i: (0, 0)),
            pl.BlockSpec((1, E), lambda i: (0, 0)),
        ],
        out_specs=pl.BlockSpec((_BM, E), lambda i: (i, 0)),
        out_shape=jax.ShapeDtypeStruct((M, E), jnp.float32),
        compiler_params=pltpu.CompilerParams(
            dimension_semantics=("parallel",),
        ),
    )(x, W.T, b.reshape(1, E))
    return out.reshape(B, S, E)


# speedup vs baseline: 1.3860x; 1.0002x over previous
import jax
import jax.numpy as jnp
from jax.experimental import pallas as pl
from jax.experimental.pallas import tpu as pltpu

_BM = 4096


def _linear_kernel(x_ref, w_ref, b_ref, o_ref):
    acc = jax.lax.dot_general(
        x_ref[...],
        w_ref[...],
        dimension_numbers=(((1,), (0,)), ((), ())),
        preferred_element_type=jnp.float32,
    )
    o_ref[...] = acc + b_ref[...]


def kernel(src, W, b):
    B, S, D = src.shape
    E = W.shape[0]
    M = B * S
    x = src.reshape(M, D)

    out = pl.pallas_call(
        _linear_kernel,
        grid=(M // _BM,),
        in_specs=[
            pl.BlockSpec((_BM, D), lambda i: (i, 0)),
            pl.BlockSpec((D, E), lambda i: (0, 0)),
            pl.BlockSpec((1, E), lambda i: (0, 0)),
        ],
        out_specs=pl.BlockSpec((_BM, E), lambda i: (i, 0)),
        out_shape=jax.ShapeDtypeStruct((M, E), jnp.float32),
        compiler_params=pltpu.CompilerParams(
            dimension_semantics=("parallel",),
        ),
    )(x, W.T, b.reshape(1, E))
    return out.reshape(B, S, E)


# speedup vs baseline: 1.3903x; 1.0031x over previous
import jax
import jax.numpy as jnp
from jax.experimental import pallas as pl
from jax.experimental.pallas import tpu as pltpu

_CH = 2048
_NBUF = 4


def _make_body(num_chunks, D, E):
    def body(x_hbm, w_hbm, b_hbm, o_hbm,
             xbuf, obuf, wv, bv, load_sem, store_sem, w_sem, b_sem):
        pltpu.make_async_copy(w_hbm, wv, w_sem).start()
        pltpu.make_async_copy(b_hbm, bv, b_sem).start()
        for i in range(min(_NBUF, num_chunks)):
            pltpu.make_async_copy(
                x_hbm.at[pl.ds(i * _CH, _CH), :], xbuf.at[i], load_sem.at[i]
            ).start()
        pltpu.make_async_copy(w_hbm, wv, w_sem).wait()
        pltpu.make_async_copy(b_hbm, bv, b_sem).wait()

        for i in range(num_chunks):
            slot = i % _NBUF
            pltpu.make_async_copy(
                x_hbm.at[pl.ds(i * _CH, _CH), :], xbuf.at[slot], load_sem.at[slot]
            ).wait()
            if i >= _NBUF:
                pltpu.make_async_copy(
                    obuf.at[slot],
                    o_hbm.at[pl.ds((i - _NBUF) * _CH, _CH), :],
                    store_sem.at[slot],
                ).wait()
            acc = jax.lax.dot_general(
                xbuf[slot], wv[...],
                dimension_numbers=(((1,), (0,)), ((), ())),
                preferred_element_type=jnp.float32,
            )
            obuf[slot] = acc + bv[...]
            pltpu.make_async_copy(
                obuf.at[slot], o_hbm.at[pl.ds(i * _CH, _CH), :], store_sem.at[slot]
            ).start()
            nxt = i + _NBUF
            if nxt < num_chunks:
                pltpu.make_async_copy(
                    x_hbm.at[pl.ds(nxt * _CH, _CH), :], xbuf.at[slot],
                    load_sem.at[slot],
                ).start()

        for i in range(max(0, num_chunks - _NBUF), num_chunks):
            slot = i % _NBUF
            pltpu.make_async_copy(
                obuf.at[slot], o_hbm.at[pl.ds(i * _CH, _CH), :], store_sem.at[slot]
            ).wait()

    return body


def kernel(src, W, b):
    B, S, D = src.shape
    E = W.shape[0]
    M = B * S
    num_chunks = M // _CH
    x = src.reshape(M, D)

    out = pl.pallas_call(
        _make_body(num_chunks, D, E),
        in_specs=[
            pl.BlockSpec(memory_space=pltpu.HBM),
            pl.BlockSpec(memory_space=pltpu.HBM),
            pl.BlockSpec(memory_space=pltpu.HBM),
        ],
        out_specs=pl.BlockSpec(memory_space=pltpu.HBM),
        out_shape=jax.ShapeDtypeStruct((M, E), jnp.float32),
        scratch_shapes=[
            pltpu.VMEM((_NBUF, _CH, D), jnp.float32),
            pltpu.VMEM((_NBUF, _CH, E), jnp.float32),
            pltpu.VMEM((D, E), jnp.float32),
            pltpu.VMEM((1, E), jnp.float32),
            pltpu.SemaphoreType.DMA((_NBUF,)),
            pltpu.SemaphoreType.DMA((_NBUF,)),
            pltpu.SemaphoreType.DMA,
            pltpu.SemaphoreType.DMA,
        ],
    )(x, W.T, b.reshape(1, E))
    return out.reshape(B, S, E)
